# Initial kernel scaffold; baseline (speedup 1.0000x reference)
import functools

import jax
import jax.numpy as jnp
from jax import lax
from jax.experimental import pallas as pl
from jax.experimental.pallas import tpu as pltpu
from jax.experimental.pallas import tpu_sc as plsc

N_CAT = 26
VOCAB = 100000
D_MODEL = 32
BATCH = 16384

NC, NS = 2, 16
NW = NC * NS
ROWS = BATCH * N_CAT
RPW = ROWS // NW
GSZ = 128
NGRP = RPW // GSZ
GPC = 4
RC = GPC * GSZ
NCHUNK = NGRP // GPC
NBUF = 2


def _body(tab, xf, off, out, x_v, off_v, rows0, rows1, gs0, gs1, os0, os1):
    wid = lax.axis_index("s") * NC + lax.axis_index("c")
    rows = (rows0, rows1)
    gsem = (gs0, gs1)
    osem = (os0, os1)

    pltpu.sync_copy(off, off_v)
    pltpu.sync_copy(xf.at[wid], x_v)

    def compute_idx(c):
        def inner(gg, carry):
            g = c * GPC + gg
            for j in range(GSZ // 16):
                sl = pl.ds(j * 16, 16)
                x_v[g, sl] = x_v[g, sl] + off_v[g, sl]
            return carry
        lax.fori_loop(0, GPC, inner, 0)

    def issue_gather(c, b):
        for t in range(GPC):
            g = c * GPC + t
            pltpu.async_copy(tab.at[x_v.at[g]],
                             rows[b].at[pl.ds(t * GSZ, GSZ)], gsem[b])

    def drain_gather(b):
        pltpu.make_async_copy(tab.at[pl.ds(0, RC)], rows[b], gsem[b]).wait()

    def issue_out(c, b):
        pltpu.async_copy(rows[b], out.at[wid].at[pl.ds(c * RC, RC)], osem[b])

    def drain_out(b):
        pltpu.make_async_copy(rows[b], out.at[0].at[pl.ds(0, RC)],
                              osem[b]).wait()

    for b in range(NBUF):
        compute_idx(b)
        issue_gather(b, b)

    def loop_body(it, carry):
        for b in range(NBUF):
            c = it * NBUF + b
            drain_gather(b)
            issue_out(c, b)
            nc = c + NBUF

            @pl.when(nc < NCHUNK)
            def _():
                compute_idx(nc)
                drain_out(b)
                issue_gather(nc, b)
        return carry

    lax.fori_loop(0, NCHUNK // NBUF, loop_body, 0)

    for b in range(NBUF):
        drain_out(b)


_lookup = functools.partial(
    pl.kernel,
    out_type=jax.ShapeDtypeStruct((NW, RPW, D_MODEL), jnp.float32),
    mesh=plsc.VectorSubcoreMesh(core_axis_name="c", subcore_axis_name="s"),
    scratch_types=[
        pltpu.VMEM((NGRP, GSZ), jnp.int32),
        pltpu.VMEM((NGRP, GSZ), jnp.int32),
        pltpu.VMEM((RC, D_MODEL), jnp.float32),
        pltpu.VMEM((RC, D_MODEL), jnp.float32),
        pltpu.SemaphoreType.DMA,
        pltpu.SemaphoreType.DMA,
        pltpu.SemaphoreType.DMA,
        pltpu.SemaphoreType.DMA,
    ],
)(_body)


def kernel(x, tables):
    xf = x.astype(jnp.int32).reshape(NW, NGRP, GSZ)
    tab = tables.reshape(N_CAT * VOCAB, D_MODEL)
    off = ((jnp.arange(RPW, dtype=jnp.int32) % N_CAT) * VOCAB).reshape(NGRP, GSZ)
    out = _lookup(tab, xf, off)
    return out.reshape(BATCH, N_CAT, D_MODEL)


# speedup vs baseline: 1.1510x; 1.1510x over previous
import functools

import jax
import jax.numpy as jnp
from jax import lax
from jax.experimental import pallas as pl
from jax.experimental.pallas import tpu as pltpu
from jax.experimental.pallas import tpu_sc as plsc

N_CAT = 26
VOCAB = 100000
D_MODEL = 32
BATCH = 16384

NC, NS = 2, 16
NW = NC * NS
ROWS = BATCH * N_CAT
RPW = ROWS // NW
GSZ = 128
NGRP = RPW // GSZ
GPC = 4
RC = GPC * GSZ
NCHUNK = NGRP // GPC
NBUF = 2


def _body(tab, xf, off, out, x_v, off_v, rows0, rows1, gs0, gs1, os0, os1):
    wid = lax.axis_index("s") * NC + lax.axis_index("c")
    rows = (rows0, rows1)
    gsem = (gs0, gs1)
    osem = (os0, os1)

    pltpu.sync_copy(off, off_v)
    pltpu.sync_copy(xf.at[wid], x_v)

    def compute_idx(c):
        def inner(gg, carry):
            g = c * GPC + gg
            for j in range(GSZ // 16):
                sl = pl.ds(j * 16, 16)
                x_v[g, sl] = x_v[g, sl] + off_v[g, sl]
            return carry
        lax.fori_loop(0, GPC, inner, 0)

    def issue_gather(c, b):
        for t in range(GPC):
            g = c * GPC + t
            pltpu.async_copy(tab.at[x_v.at[g]],
                             rows[b].at[pl.ds(t * GSZ, GSZ)], gsem[b])

    def drain_gather(b):
        pltpu.make_async_copy(tab.at[pl.ds(0, RC)], rows[b], gsem[b]).wait()

    def issue_out(c, b):
        pltpu.async_copy(rows[b], out.at[wid].at[pl.ds(c * RC, RC)], osem[b])

    def drain_out(b):
        pltpu.make_async_copy(rows[b], out.at[0].at[pl.ds(0, RC)],
                              osem[b]).wait()

    for b in range(NBUF):
        compute_idx(b)
        issue_gather(b, b)

    def loop_body(it, carry):
        for b in range(NBUF):
            c = it * NBUF + b
            drain_gather(b)
            issue_out(c, b)
            nc = c + NBUF

            @pl.when(nc < NCHUNK)
            def _():
                compute_idx(nc)
                drain_out(b)
                issue_gather(nc, b)
        return carry

    lax.fori_loop(0, NCHUNK // NBUF, loop_body, 0)

    for b in range(NBUF):
        drain_out(b)


_lookup = functools.partial(
    pl.kernel,
    out_type=jax.ShapeDtypeStruct((NW, RPW, D_MODEL), jnp.float32),
    mesh=plsc.VectorSubcoreMesh(core_axis_name="c", subcore_axis_name="s"),
    compiler_params=pltpu.CompilerParams(use_tc_tiling_on_sc=False),
    scratch_types=[
        pltpu.VMEM((NGRP, GSZ), jnp.int32),
        pltpu.VMEM((NGRP, GSZ), jnp.int32),
        pltpu.VMEM((RC, D_MODEL), jnp.float32),
        pltpu.VMEM((RC, D_MODEL), jnp.float32),
        pltpu.SemaphoreType.DMA,
        pltpu.SemaphoreType.DMA,
        pltpu.SemaphoreType.DMA,
        pltpu.SemaphoreType.DMA,
    ],
)(_body)


def kernel(x, tables):
    xf = x.astype(jnp.int32).reshape(NW, NGRP, GSZ)
    tab = tables.reshape(N_CAT * VOCAB, D_MODEL)
    off = ((jnp.arange(RPW, dtype=jnp.int32) % N_CAT) * VOCAB).reshape(NGRP, GSZ)
    out = _lookup(tab, xf, off)
    return out.reshape(BATCH, N_CAT, D_MODEL)


# speedup vs baseline: 1.1669x; 1.0139x over previous
import functools

import jax
import jax.numpy as jnp
from jax import lax
from jax.experimental import pallas as pl
from jax.experimental.pallas import tpu as pltpu
from jax.experimental.pallas import tpu_sc as plsc

N_CAT = 26
VOCAB = 100000
D_MODEL = 32
BATCH = 16384

NC, NS = 2, 16
NW = NC * NS
NU = N_CAT * (BATCH // 128) // NW
NBUF = 2


def _body(tab, xcol, out5, x_v, idx0, idx1, rows0, rows1, tb0, tb1,
          gs0, gs1, os0, os1):
    wid = lax.axis_index("s") * NC + lax.axis_index("c")
    idxb = (idx0, idx1)
    rows = (rows0, rows1)
    tbuf = (tb0, tb1)
    gsem = (gs0, gs1)
    osem = (os0, os1)

    pltpu.sync_copy(xcol.at[pl.ds(wid * NU, NU)], x_v)

    lanes = lax.iota(jnp.int32, 16)
    tidx0 = lanes * 128
    tidx1 = tidx0 + 16 * 128

    def prep(u, b):
        g = wid * NU + u
        i = lax.shift_right_logical(g, 7)
        base = i * VOCAB

        def inner(j, carry):
            sl = pl.ds(j * 16, 16)
            idxb[b][sl] = x_v[u, sl] + base
            return carry
        lax.fori_loop(0, 8, inner, 0)

    def issue_gather(b):
        pltpu.async_copy(tab.at[idxb[b]], rows[b], gsem[b])

    def drain_gather(b):
        pltpu.make_async_copy(tab.at[idxb[b]], rows[b], gsem[b]).wait()

    def transpose(b):
        def inner(pp, carry):
            for p2 in range(4):
                p = pp * 4 + p2
                lo = rows[b][p, pl.ds(0, 16)]
                plsc.store_scatter(tbuf[b], [tidx0 + p], lo)
                hi = rows[b][p, pl.ds(16, 16)]
                plsc.store_scatter(tbuf[b], [tidx1 + p], hi)
            return carry
        lax.fori_loop(0, 32, inner, 0)

    def issue_out(u, b):
        g = wid * NU + u
        i = lax.shift_right_logical(g, 7)
        tb = lax.bitwise_and(g, 127)
        for td in range(4):
            pltpu.async_copy(tbuf[b].at[pl.ds(td * 1024, 1024)],
                             out5.at[i, td, tb], osem[b])

    def drain_out(b):
        for td in range(4):
            pltpu.make_async_copy(tbuf[b].at[pl.ds(td * 1024, 1024)],
                                  out5.at[0, 0, 0], osem[b]).wait()

    for b in range(NBUF):
        prep(b, b)
        issue_gather(b)

    NITER = NU // NBUF

    def loop_body(it, carry):
        for b in range(NBUF):
            u = it * NBUF + b
            drain_gather(b)

            @pl.when(it >= 1)
            def _():
                drain_out(b)
            transpose(b)

            @pl.when(it < NITER - 1)
            def _():
                prep(u + NBUF, b)
                issue_gather(b)
            issue_out(u, b)
        return carry

    lax.fori_loop(0, NITER, loop_body, 0)

    for b in range(NBUF):
        drain_out(b)


_lookup = functools.partial(
    pl.kernel,
    out_type=jax.ShapeDtypeStruct((N_CAT, 4, 128, 1024), jnp.float32),
    mesh=plsc.VectorSubcoreMesh(core_axis_name="c", subcore_axis_name="s"),
    compiler_params=pltpu.CompilerParams(use_tc_tiling_on_sc=False,
                                         needs_layout_passes=False),
    scratch_types=[
        pltpu.VMEM((NU, 128), jnp.int32),
        pltpu.VMEM((128,), jnp.int32),
        pltpu.VMEM((128,), jnp.int32),
        pltpu.VMEM((128, D_MODEL), jnp.float32),
        pltpu.VMEM((128, D_MODEL), jnp.float32),
        pltpu.VMEM((D_MODEL * 128,), jnp.float32),
        pltpu.VMEM((D_MODEL * 128,), jnp.float32),
        pltpu.SemaphoreType.DMA,
        pltpu.SemaphoreType.DMA,
        pltpu.SemaphoreType.DMA,
        pltpu.SemaphoreType.DMA,
    ],
)(_body)


def kernel(x, tables):
    xcol = x.astype(jnp.int32).T.reshape(N_CAT * BATCH // 128, 128)
    tab = tables.reshape(N_CAT * VOCAB, D_MODEL)
    out5 = _lookup(tab, xcol).reshape(N_CAT, 4, 128, 8, 128)
    return out5.transpose(2, 4, 0, 1, 3).reshape(BATCH, N_CAT, D_MODEL)


# speedup vs baseline: 1.3105x; 1.1230x over previous
import functools

import jax
import jax.numpy as jnp
from jax import lax
from jax.experimental import pallas as pl
from jax.experimental.pallas import tpu as pltpu
from jax.experimental.pallas import tpu_sc as plsc

N_CAT = 26
VOCAB = 100000
D_MODEL = 32
BATCH = 16384

NC, NS = 2, 16
NW = NC * NS
TPW = (BATCH // 128) // NW
NU = N_CAT * TPW
NBUF = 2
PITCH = 129


def _body(tab, xT, out5, x_v, idx0, idx1, rows0, rows1, tb0, tb1,
          gs0, gs1, os0, os1):
    wid = lax.axis_index("s") * NC + lax.axis_index("c")
    idxb = (idx0, idx1)
    rows = (rows0, rows1)
    tbuf = (tb0, tb1)
    gsem = (gs0, gs1)
    osem = (os0, os1)

    pltpu.sync_copy(xT.at[:, pl.ds(wid * (TPW * 128), TPW * 128)], x_v)

    lanes = lax.iota(jnp.int32, 16)
    hi = lanes + 16

    def prep(u, b):
        i = lax.shift_right_logical(u, 2)
        t = lax.bitwise_and(u, 3)
        base = i * VOCAB

        def inner(j, carry):
            idxb[b][pl.ds(j * 16, 16)] = (
                x_v[i, pl.ds(t * 128 + j * 16, 16)] + base)
            return carry
        lax.fori_loop(0, 8, inner, 0)

    def issue_gather(b):
        pltpu.async_copy(tab.at[idxb[b]], rows[b], gsem[b])

    def drain_gather(b):
        pltpu.make_async_copy(tab.at[idxb[b]], rows[b], gsem[b]).wait()

    def transpose(b):
        def inner(pp, carry):
            for p2 in range(4):
                p = pp * 4 + p2
                pvec = lanes * 0 + p
                plsc.store_scatter(tbuf[b], [lanes, pvec],
                                   rows[b][p, pl.ds(0, 16)])
                plsc.store_scatter(tbuf[b], [hi, pvec],
                                   rows[b][p, pl.ds(16, 16)])
            return carry
        lax.fori_loop(0, 32, inner, 0)

    def issue_out(u, b):
        i = lax.shift_right_logical(u, 2)
        t = lax.bitwise_and(u, 3)
        tb = wid * TPW + t
        for td in range(4):
            pltpu.async_copy(
                tbuf[b].at[pl.ds(td * 8, 8), pl.ds(0, 128)],
                out5.at[i, td, tb], osem[b])

    def drain_out(b):
        for td in range(4):
            pltpu.make_async_copy(
                tbuf[b].at[pl.ds(td * 8, 8), pl.ds(0, 128)],
                out5.at[0, 0, 0], osem[b]).wait()

    for b in range(NBUF):
        prep(b, b)
        issue_gather(b)

    NITER = NU // NBUF

    def loop_body(it, carry):
        for b in range(NBUF):
            u = it * NBUF + b
            drain_gather(b)

            @pl.when(it >= 1)
            def _():
                drain_out(b)
            transpose(b)

            @pl.when(it < NITER - 1)
            def _():
                prep(u + NBUF, b)
                issue_gather(b)
            issue_out(u, b)
        return carry

    lax.fori_loop(0, NITER, loop_body, 0)

    for b in range(NBUF):
        drain_out(b)


_lookup = functools.partial(
    pl.kernel,
    out_type=jax.ShapeDtypeStruct((N_CAT, 4, 128, 8, 128), jnp.float32),
    mesh=plsc.VectorSubcoreMesh(core_axis_name="c", subcore_axis_name="s"),
    compiler_params=pltpu.CompilerParams(use_tc_tiling_on_sc=False,
                                         needs_layout_passes=False),
    scratch_types=[
        pltpu.VMEM((N_CAT, TPW * 128), jnp.int32),
        pltpu.VMEM((128,), jnp.int32),
        pltpu.VMEM((128,), jnp.int32),
        pltpu.VMEM((128, D_MODEL), jnp.float32),
        pltpu.VMEM((128, D_MODEL), jnp.float32),
        pltpu.VMEM((D_MODEL, PITCH), jnp.float32),
        pltpu.VMEM((D_MODEL, PITCH), jnp.float32),
        pltpu.SemaphoreType.DMA,
        pltpu.SemaphoreType.DMA,
        pltpu.SemaphoreType.DMA,
        pltpu.SemaphoreType.DMA,
    ],
)(_body)


def kernel(x, tables):
    xT = x.astype(jnp.int32).T
    tab = tables.reshape(N_CAT * VOCAB, D_MODEL)
    out5 = _lookup(tab, xT)
    return out5.transpose(2, 4, 0, 1, 3).reshape(BATCH, N_CAT, D_MODEL)


# speedup vs baseline: 2.0781x; 1.5857x over previous
import functools

import jax
import jax.numpy as jnp
from jax import lax
from jax.experimental import pallas as pl
from jax.experimental.pallas import tpu as pltpu
from jax.experimental.pallas import tpu_sc as plsc

N_CAT = 26
VOCAB = 100000
D_MODEL = 32
BATCH = 16384

NC, NS = 2, 16
NW = NC * NS
NPL = N_CAT * D_MODEL
PPW = NPL // NW
BCH = 4096
NCH = BATCH // BCH


def _body(tT, xT, out5, plane_v, xc_v, blk_v, psem, xsem, osem):
    wid = lax.axis_index("s") * NC + lax.axis_index("c")
    base = wid * PPW

    def plane_ref(su):
        i = lax.div(su, D_MODEL)
        d = lax.rem(su, D_MODEL)
        return tT.at[i, d]

    def issue_plane(su):
        pltpu.async_copy(plane_ref(su), plane_v, psem)

    def wait_plane(su):
        pltpu.make_async_copy(plane_ref(su), plane_v, psem).wait()

    def out_ref(su):
        i = lax.div(su, D_MODEL)
        d = lax.rem(su, D_MODEL)
        td = lax.shift_right_logical(d, 3)
        dr = lax.bitwise_and(d, 7)
        return out5.at[i, td, pl.ds(0, 128), pl.ds(dr, 1), pl.ds(0, 128)]

    def issue_out(su):
        pltpu.async_copy(blk_v, out_ref(su), osem)

    def drain_out(su):
        pltpu.make_async_copy(blk_v, out_ref(su), osem).wait()

    def gather_plane(su):
        i = lax.div(su, D_MODEL)
        for c in range(NCH):
            pltpu.sync_copy(xT.at[i, pl.ds(c * BCH, BCH)], xc_v)

            def inner(f, carry):
                for k in range(8):
                    v = (c * (BCH // 16)) + f * 8 + k
                    idx = xc_v[pl.ds((f * 8 + k) * 16, 16)]
                    vals = plsc.load_gather(plane_v, [idx])
                    tb = lax.shift_right_logical(v, 3)
                    bc = lax.bitwise_and(v, 7) * 16
                    blk_v[tb, 0, pl.ds(bc, 16)] = vals
                return carry
            lax.fori_loop(0, BCH // 128, inner, 0)

    issue_plane(base)

    def loop_body(s, carry):
        su = base + s
        wait_plane(su)

        @pl.when(s >= 1)
        def _():
            drain_out(su - 1)
        gather_plane(su)

        @pl.when(s < PPW - 1)
        def _():
            issue_plane(su + 1)
        issue_out(su)
        return carry

    lax.fori_loop(0, PPW, loop_body, 0)
    drain_out(base + PPW - 1)


_lookup = functools.partial(
    pl.kernel,
    out_type=jax.ShapeDtypeStruct((N_CAT, 4, 128, 8, 128), jnp.float32),
    mesh=plsc.VectorSubcoreMesh(core_axis_name="c", subcore_axis_name="s"),
    compiler_params=pltpu.CompilerParams(use_tc_tiling_on_sc=False,
                                         needs_layout_passes=False),
    scratch_types=[
        pltpu.VMEM((VOCAB,), jnp.float32),
        pltpu.VMEM((BCH,), jnp.int32),
        pltpu.VMEM((128, 1, 128), jnp.float32),
        pltpu.SemaphoreType.DMA,
        pltpu.SemaphoreType.DMA,
        pltpu.SemaphoreType.DMA,
    ],
)(_body)


def kernel(x, tables):
    tT = tables.transpose(0, 2, 1)
    xT = x.astype(jnp.int32).T
    out5 = _lookup(tT, xT)
    return out5.transpose(2, 4, 0, 1, 3).reshape(BATCH, N_CAT, D_MODEL)


# speedup vs baseline: 2.2867x; 1.1004x over previous
import functools

import jax
import jax.numpy as jnp
from jax import lax
from jax.experimental import pallas as pl
from jax.experimental.pallas import tpu as pltpu
from jax.experimental.pallas import tpu_sc as plsc

N_CAT = 26
VOCAB = 100000
D_MODEL = 32
BATCH = 16384

NC, NS = 2, 16
NW = NC * NS
NPL = N_CAT * D_MODEL
PPW = NPL // NW
BCH = 4096
NCH = BATCH // BCH


def _body(tT, xT, out5, plane_v, xc_v, blk_v, psem, xsem, osem):
    wid = lax.axis_index("s") * NC + lax.axis_index("c")
    base = wid * PPW

    def plane_ref(su):
        i = lax.div(su, D_MODEL)
        d = lax.rem(su, D_MODEL)
        return tT.at[i, d]

    def issue_plane(su):
        pltpu.async_copy(plane_ref(su), plane_v, psem)

    def wait_plane(su):
        pltpu.make_async_copy(plane_ref(su), plane_v, psem).wait()

    def out_ref(su, h):
        i = lax.div(su, D_MODEL)
        d = lax.rem(su, D_MODEL)
        td = lax.shift_right_logical(d, 3)
        dr = lax.bitwise_and(d, 7)
        return out5.at[i, td, pl.ds(h * 64, 64), pl.ds(dr, 1), pl.ds(0, 128)]

    def issue_out(su, h):
        pltpu.async_copy(blk_v, out_ref(su, h), osem)

    def drain_out():
        pltpu.make_async_copy(
            blk_v, out5.at[0, 0, pl.ds(0, 64), pl.ds(0, 1), pl.ds(0, 128)],
            osem).wait()

    def gather_half(h):
        def inner(f, carry):
            for k in range(8):
                idx = xc_v[pl.ds(h * 8192 + f * 128 + k * 16, 16)]
                vals = plsc.load_gather(plane_v, [idx])
                blk_v[f, 0, pl.ds(k * 16, 16)] = vals
            return carry
        lax.fori_loop(0, 64, inner, 0)

    issue_plane(base)

    def loop_body(s, i_prev):
        su = base + s
        i = lax.div(su, D_MODEL)
        wait_plane(su)

        @pl.when(i != i_prev)
        def _():
            pltpu.sync_copy(xT.at[i], xc_v)

        @pl.when(s >= 1)
        def _():
            drain_out()
        gather_half(0)
        issue_out(su, 0)
        drain_out()
        gather_half(1)

        @pl.when(s < PPW - 1)
        def _():
            issue_plane(su + 1)
        issue_out(su, 1)
        return i

    lax.fori_loop(0, PPW, loop_body, jnp.int32(-1))
    drain_out()


_lookup = functools.partial(
    pl.kernel,
    out_type=jax.ShapeDtypeStruct((N_CAT, 4, 128, 8, 128), jnp.float32),
    mesh=plsc.VectorSubcoreMesh(core_axis_name="c", subcore_axis_name="s"),
    compiler_params=pltpu.CompilerParams(use_tc_tiling_on_sc=False,
                                         needs_layout_passes=False),
    scratch_types=[
        pltpu.VMEM((VOCAB,), jnp.float32),
        pltpu.VMEM((BATCH,), jnp.int32),
        pltpu.VMEM((64, 1, 128), jnp.float32),
        pltpu.SemaphoreType.DMA,
        pltpu.SemaphoreType.DMA,
        pltpu.SemaphoreType.DMA,
    ],
)(_body)


def kernel(x, tables):
    tT = tables.transpose(0, 2, 1)
    xT = x.astype(jnp.int32).T
    out5 = _lookup(tT, xT)
    return out5.transpose(2, 4, 0, 1, 3).reshape(BATCH, N_CAT, D_MODEL)


# speedup vs baseline: 6.2069x; 2.7143x over previous
import functools

import jax
import jax.numpy as jnp
from jax import lax
from jax.experimental import pallas as pl
from jax.experimental.pallas import tpu as pltpu
from jax.experimental.pallas import tpu_sc as plsc

N_CAT = 26
VOCAB = 100000
D_MODEL = 32
BATCH = 16384

NC, NS = 2, 16
NW = NC * NS
NPL = N_CAT * D_MODEL
PPW = NPL // NW
BCH = 4096
NCH = BATCH // BCH


def _body(tT, xT, out5, plane_v, xc_v, blk_v, psem, xsem, osem):
    wid = lax.axis_index("s") * NC + lax.axis_index("c")
    base = wid * PPW

    def plane_ref(su):
        i = lax.div(su, D_MODEL)
        d = lax.rem(su, D_MODEL)
        return tT.at[i, d]

    def issue_plane(su):
        pltpu.async_copy(plane_ref(su), plane_v, psem)

    def wait_plane(su):
        pltpu.make_async_copy(plane_ref(su), plane_v, psem).wait()

    def out_ref(su, h):
        i = lax.div(su, D_MODEL)
        d = lax.rem(su, D_MODEL)
        td = lax.shift_right_logical(d, 3)
        dr = lax.bitwise_and(d, 7)
        return out5.at[i, td, pl.ds(h * 64, 64), pl.ds(dr, 1), pl.ds(0, 128)]

    def issue_out(su, h):
        pltpu.async_copy(blk_v, out_ref(su, h), osem)

    def drain_out():
        pltpu.make_async_copy(
            blk_v, out5.at[0, 0, pl.ds(0, 64), pl.ds(0, 1), pl.ds(0, 128)],
            osem).wait()

    def gather_half(h):
        def inner(f, carry):
            for k in range(8):
                idx = xc_v[pl.ds(h * 8192 + f * 128 + k * 16, 16)]
                vals = plsc.load_gather(plane_v, [idx])
                blk_v[f, 0, pl.ds(k * 16, 16)] = vals
            return carry
        lax.fori_loop(0, 64, inner, 0)

    issue_plane(base)

    def loop_body(s, i_prev):
        su = base + s
        i = lax.div(su, D_MODEL)
        wait_plane(su)

        @pl.when(i != i_prev)
        def _():
            pltpu.sync_copy(xT.at[i], xc_v)

        @pl.when(s >= 1)
        def _():
            drain_out()
        gather_half(0)
        issue_out(su, 0)
        drain_out()
        gather_half(1)

        @pl.when(s < PPW - 1)
        def _():
            issue_plane(su + 1)
        issue_out(su, 1)
        return i

    lax.fori_loop(0, PPW, loop_body, jnp.int32(-1))
    drain_out()


_lookup = functools.partial(
    pl.kernel,
    out_type=jax.ShapeDtypeStruct((N_CAT, 4, 128, 8, 128), jnp.float32),
    mesh=plsc.VectorSubcoreMesh(core_axis_name="c", subcore_axis_name="s"),
    compiler_params=pltpu.CompilerParams(needs_layout_passes=False),
    scratch_types=[
        pltpu.VMEM((VOCAB,), jnp.float32),
        pltpu.VMEM((BATCH,), jnp.int32),
        pltpu.VMEM((64, 1, 128), jnp.float32),
        pltpu.SemaphoreType.DMA,
        pltpu.SemaphoreType.DMA,
        pltpu.SemaphoreType.DMA,
    ],
)(_body)


def kernel(x, tables):
    tT = tables.transpose(0, 2, 1)
    xT = x.astype(jnp.int32).T
    out5 = _lookup(tT, xT)
    return out5.transpose(2, 4, 0, 1, 3).reshape(BATCH, N_CAT, D_MODEL)
